# baseline (speedup 1.0000x reference)
import functools

import jax
import jax.numpy as jnp
from jax import lax
from jax.experimental import pallas as pl
from jax.experimental.pallas import tpu as pltpu
from jax.experimental.pallas import tpu_sc as plsc

N = 10000
E = 320000
D = 128
BN_EPS = 1e-5

NC = 2
NS = 16
NW = NC * NS
NP = 10240
RPT = NP // NS
PAD_ROW = NP - 8

EPW = E // NW
CH = 80
NCH = 126
EPWP = NCH * CH
NBUF = 2


def _make_agg():
    mesh = plsc.VectorSubcoreMesh(core_axis_name="c", subcore_axis_name="s")

    @functools.partial(
        pl.kernel,
        out_type=jax.ShapeDtypeStruct((NC, NP, D), jnp.float32),
        mesh=mesh,
        scratch_types=[
            pltpu.VMEM_SHARED((NP, D), jnp.float32),
            pltpu.VMEM((EPWP,), jnp.int32),
            pltpu.VMEM((NCH, CH), jnp.int32),
            [pltpu.VMEM((CH, D), jnp.float32)] * NBUF,
            [pltpu.SemaphoreType.DMA] * NBUF,
            pltpu.SemaphoreType.DMA,
        ],
    )
    def agg(h_hbm, z_hbm, src_hbm, dst_hbm, out_hbm,
            shared, sidx, didx, rows, sem_g, sem_i):
        c = lax.axis_index("c")
        s = lax.axis_index("s")
        wid = s * NC + c

        pltpu.async_copy(src_hbm.at[wid], sidx, sem_i)
        pltpu.async_copy(dst_hbm.at[wid], didx, sem_i)

        rstart = s * RPT

        @pl.when(c == 0)
        def _():
            pltpu.sync_copy(h_hbm.at[pl.ds(rstart, RPT)],
                            shared.at[pl.ds(rstart, RPT)])

        @pl.when(c == 1)
        def _():
            pltpu.sync_copy(z_hbm.at[pl.ds(rstart, RPT)],
                            shared.at[pl.ds(rstart, RPT)])

        pltpu.make_async_copy(src_hbm.at[wid], sidx, sem_i).wait()
        pltpu.make_async_copy(dst_hbm.at[wid], didx, sem_i).wait()

        def fire_g(chunk, b):
            pltpu.async_copy(h_hbm.at[sidx.at[pl.ds(chunk * CH, CH)]],
                             rows[b], sem_g[b])

        def finish(chunk, b):
            pltpu.make_async_copy(h_hbm.at[sidx.at[pl.ds(chunk * CH, CH)]],
                                  rows[b], sem_g[b]).wait()
            pltpu.sync_copy(rows[b], shared.at[didx.at[chunk]], add=True)

        fire_g(0, 0)
        plsc.subcore_barrier()
        fire_g(1, 1)

        def body(io, carry):
            c0 = 2 * io
            finish(c0, 0)

            @pl.when(c0 + 2 < NCH)
            def _():
                fire_g(c0 + 2, 0)

            finish(c0 + 1, 1)

            @pl.when(c0 + 3 < NCH)
            def _():
                fire_g(c0 + 3, 1)

            return carry

        lax.fori_loop(0, NCH // 2, body, 0)
        plsc.subcore_barrier()

        pltpu.sync_copy(shared.at[pl.ds(rstart, RPT)],
                        out_hbm.at[c].at[pl.ds(rstart, RPT)])

    return agg


_agg = _make_agg()


def _bn_stats(t):
    rid = lax.broadcasted_iota(jnp.int32, (NP, 1), 0)
    m = (rid < N).astype(jnp.float32)
    tm = t * m
    s1 = jnp.sum(tm, axis=0, keepdims=True)
    s2 = jnp.sum(tm * tm, axis=0, keepdims=True)
    mu = s1 / N
    var = s2 / N - mu * mu
    return m, mu, lax.rsqrt(var + BN_EPS)


def _dense_body(p_ref, w_ref, b_ref, g_ref, be_ref, o_ref):
    u = p_ref[0] + p_ref[1]
    t = jnp.dot(u, w_ref[...], preferred_element_type=jnp.float32) + b_ref[...]
    m, mu, rstd = _bn_stats(t)
    o_ref[...] = jnp.maximum(g_ref[...] * (t - mu) * rstd + be_ref[...],
                             0.0) * m


def _dense3_body(p_ref, w_ref, b_ref, g_ref, be_ref, w4_ref, b4_ref, o_ref):
    u = p_ref[0] + p_ref[1]
    t = jnp.dot(u, w_ref[...], preferred_element_type=jnp.float32) + b_ref[...]
    m, mu, rstd = _bn_stats(t)
    y = jnp.maximum(g_ref[...] * (t - mu) * rstd + be_ref[...], 0.0)
    o_ref[...] = (jnp.dot(y, w4_ref[...], preferred_element_type=jnp.float32)
                  + b4_ref[...])


def _dense(p, W, b, g, be):
    return pl.pallas_call(
        _dense_body,
        out_shape=jax.ShapeDtypeStruct((NP, D), jnp.float32),
    )(p, W, b.reshape(1, D), g.reshape(1, D), be.reshape(1, D))


def _dense3(p, W, b, g, be, W4, b4):
    return pl.pallas_call(
        _dense3_body,
        out_shape=jax.ShapeDtypeStruct((NP, D), jnp.float32),
    )(p, W, b.reshape(1, D), g.reshape(1, D), be.reshape(1, D),
      W4, b4.reshape(1, D))


def kernel(x, adj_t, W1, b1, g1, be1, W2, b2, g2, be2, W3, b3, g3, be3, W4, b4):
    npad = EPWP - EPW
    dpad = N + (jnp.arange(NW * npad, dtype=jnp.int32) % (NP - N - 8))
    src = jnp.pad(adj_t[0].reshape(NW, EPW), ((0, 0), (0, npad)),
                  constant_values=PAD_ROW)
    dst = jnp.concatenate(
        [adj_t[1].reshape(NW, EPW), dpad.reshape(NW, npad)],
        axis=1).reshape(NW, NCH, CH)
    xp = jnp.pad(x, ((0, NP - N), (0, 0)))
    z = jnp.zeros((NP, D), jnp.float32)
    p = _agg(xp, z, src, dst)
    h = _dense(p, W1, b1, g1, be1)
    p = _agg(h, z, src, dst)
    h = _dense(p, W2, b2, g2, be2)
    p = _agg(h, z, src, dst)
    out = _dense3(p, W3, b3, g3, be3, W4, b4)
    return out[:N]


# speedup vs baseline: 1.6425x; 1.6425x over previous
import functools

import jax
import jax.numpy as jnp
from jax import lax
from jax.experimental import pallas as pl
from jax.experimental.pallas import tpu as pltpu
from jax.experimental.pallas import tpu_sc as plsc

N = 10000
E = 320000
D = 128
BN_EPS = 1e-5

NC = 2
NS = 16
NW = NC * NS
NP = 10240
RPT = NP // NS
PAD_ROW = NP - 8

EPW = E // NW
CH = 80
NCH = 126
EPWP = NCH * CH
NBUF = 2


def _make_agg():
    mesh = plsc.VectorSubcoreMesh(core_axis_name="c", subcore_axis_name="s")

    @functools.partial(
        pl.kernel,
        out_type=jax.ShapeDtypeStruct((NC, NP, D), jnp.float32),
        mesh=mesh,
        scratch_types=[
            pltpu.VMEM_SHARED((NP, D), jnp.float32),
            pltpu.VMEM((EPWP,), jnp.int32),
            pltpu.VMEM((NCH, CH), jnp.int32),
            [pltpu.VMEM((CH, D), jnp.float32)] * NBUF,
            [pltpu.SemaphoreType.DMA] * NBUF,
            pltpu.SemaphoreType.DMA,
        ],
    )
    def agg(h_hbm, z_hbm, src_hbm, dst_hbm, out_hbm,
            shared, sidx, didx, rows, sem_g, sem_i):
        c = lax.axis_index("c")
        s = lax.axis_index("s")
        wid = s * NC + c

        pltpu.async_copy(src_hbm.at[wid], sidx, sem_i)
        pltpu.async_copy(dst_hbm.at[wid], didx, sem_i)

        rstart = s * RPT

        @pl.when(c == 0)
        def _():
            pltpu.sync_copy(h_hbm.at[pl.ds(rstart, RPT)],
                            shared.at[pl.ds(rstart, RPT)])

        @pl.when(c == 1)
        def _():
            pltpu.sync_copy(z_hbm.at[pl.ds(rstart, RPT)],
                            shared.at[pl.ds(rstart, RPT)])

        pltpu.make_async_copy(src_hbm.at[wid], sidx, sem_i).wait()
        pltpu.make_async_copy(dst_hbm.at[wid], didx, sem_i).wait()

        def fire_g(chunk, b):
            pltpu.async_copy(h_hbm.at[sidx.at[pl.ds(chunk * CH, CH)]],
                             rows[b], sem_g[b])

        def finish(chunk, b):
            pltpu.make_async_copy(h_hbm.at[sidx.at[pl.ds(chunk * CH, CH)]],
                                  rows[b], sem_g[b]).wait()
            pltpu.sync_copy(rows[b], shared.at[didx.at[chunk]], add=True)

        fire_g(0, 0)
        plsc.subcore_barrier()
        fire_g(1, 1)

        def body(io, carry):
            c0 = 2 * io
            finish(c0, 0)

            @pl.when(c0 + 2 < NCH)
            def _():
                fire_g(c0 + 2, 0)

            finish(c0 + 1, 1)

            @pl.when(c0 + 3 < NCH)
            def _():
                fire_g(c0 + 3, 1)

            return carry

        lax.fori_loop(0, NCH // 2, body, 0)
        plsc.subcore_barrier()

        pltpu.sync_copy(shared.at[pl.ds(rstart, RPT)],
                        out_hbm.at[c].at[pl.ds(rstart, RPT)])

    return agg


_agg = _make_agg()


def _bn_stats(t):
    rid = lax.broadcasted_iota(jnp.int32, (NP, 1), 0)
    m = (rid < N).astype(jnp.float32)
    tm = t * m
    s1 = jnp.sum(tm, axis=0, keepdims=True)
    s2 = jnp.sum(tm * tm, axis=0, keepdims=True)
    mu = s1 / N
    var = s2 / N - mu * mu
    return m, mu, lax.rsqrt(var + BN_EPS)


def _dense_body(p_ref, w_ref, b_ref, g_ref, be_ref, o_ref):
    u = p_ref[0] + p_ref[1]
    t = jnp.dot(u, w_ref[...], preferred_element_type=jnp.float32) + b_ref[...]
    m, mu, rstd = _bn_stats(t)
    o_ref[...] = jnp.maximum(g_ref[...] * (t - mu) * rstd + be_ref[...],
                             0.0) * m


def _dense3_body(p_ref, w_ref, b_ref, g_ref, be_ref, w4_ref, b4_ref, o_ref):
    u = p_ref[0] + p_ref[1]
    t = jnp.dot(u, w_ref[...], preferred_element_type=jnp.float32) + b_ref[...]
    m, mu, rstd = _bn_stats(t)
    y = jnp.maximum(g_ref[...] * (t - mu) * rstd + be_ref[...], 0.0)
    o_ref[...] = (jnp.dot(y, w4_ref[...], preferred_element_type=jnp.float32)
                  + b4_ref[...])


def _dense(p, W, b, g, be):
    return pl.pallas_call(
        _dense_body,
        out_shape=jax.ShapeDtypeStruct((NP, D), jnp.float32),
    )(p, W, b.reshape(1, D), g.reshape(1, D), be.reshape(1, D))


def _dense3(p, W, b, g, be, W4, b4):
    return pl.pallas_call(
        _dense3_body,
        out_shape=jax.ShapeDtypeStruct((NP, D), jnp.float32),
    )(p, W, b.reshape(1, D), g.reshape(1, D), be.reshape(1, D),
      W4, b4.reshape(1, D))


def kernel(x, adj_t, W1, b1, g1, be1, W2, b2, g2, be2, W3, b3, g3, be3, W4, b4):
    npad = EPWP - EPW
    dpad = N + (jnp.arange(NW * npad, dtype=jnp.int32) % (NP - N - 8))
    src = jnp.concatenate(
        [adj_t[0].reshape(NW, EPW), dpad.reshape(NW, npad)], axis=1)
    dst = jnp.concatenate(
        [adj_t[1].reshape(NW, EPW), dpad.reshape(NW, npad)],
        axis=1).reshape(NW, NCH, CH)
    xp = jnp.pad(x, ((0, NP - N), (0, 0)))
    z = jnp.zeros((NP, D), jnp.float32)
    p = _agg(xp, z, src, dst)
    h = _dense(p, W1, b1, g1, be1)
    p = _agg(h, z, src, dst)
    h = _dense(p, W2, b2, g2, be2)
    p = _agg(h, z, src, dst)
    out = _dense3(p, W3, b3, g3, be3, W4, b4)
    return out[:N]


# speedup vs baseline: 1.6544x; 1.0072x over previous
import functools

import jax
import jax.numpy as jnp
from jax import lax
from jax.experimental import pallas as pl
from jax.experimental.pallas import tpu as pltpu
from jax.experimental.pallas import tpu_sc as plsc

N = 10000
E = 320000
D = 128
BN_EPS = 1e-5

NC = 2
NS = 16
NW = NC * NS
NP = 10240
RPT = NP // NS
PAD_ROW = NP - 8

EPW = E // NW
CH = 72
NCH = 140
EPWP = NCH * CH
NBUF = 4


def _make_agg():
    mesh = plsc.VectorSubcoreMesh(core_axis_name="c", subcore_axis_name="s")

    @functools.partial(
        pl.kernel,
        out_type=jax.ShapeDtypeStruct((NC, NP, D), jnp.float32),
        mesh=mesh,
        scratch_types=[
            pltpu.VMEM_SHARED((NP, D), jnp.float32),
            pltpu.VMEM((EPWP,), jnp.int32),
            [pltpu.VMEM((CH,), jnp.int32)] * NBUF,
            [pltpu.VMEM((CH, D), jnp.float32)] * NBUF,
            [pltpu.SemaphoreType.DMA] * NBUF,
            [pltpu.SemaphoreType.DMA] * NBUF,
            [pltpu.SemaphoreType.DMA] * NBUF,
            pltpu.SemaphoreType.DMA,
        ],
    )
    def agg(h_hbm, z_hbm, src_hbm, dst_hbm, out_hbm,
            shared, sidx, didx, rows, sem_g, sem_s, sem_d, sem_i):
        c = lax.axis_index("c")
        s = lax.axis_index("s")
        wid = s * NC + c

        pltpu.async_copy(src_hbm.at[wid], sidx, sem_i)

        rstart = s * RPT

        @pl.when(c == 0)
        def _():
            pltpu.sync_copy(h_hbm.at[pl.ds(rstart, RPT)],
                            shared.at[pl.ds(rstart, RPT)])

        @pl.when(c == 1)
        def _():
            pltpu.sync_copy(z_hbm.at[pl.ds(rstart, RPT)],
                            shared.at[pl.ds(rstart, RPT)])

        pltpu.make_async_copy(src_hbm.at[wid], sidx, sem_i).wait()

        def fire_d(chunk, b):
            pltpu.async_copy(dst_hbm.at[wid, chunk], didx[b], sem_d[b])

        def fire_g(chunk, b):
            pltpu.async_copy(h_hbm.at[sidx.at[pl.ds(chunk * CH, CH)]],
                             rows[b], sem_g[b])

        def wait_g(chunk, b):
            pltpu.make_async_copy(h_hbm.at[sidx.at[pl.ds(chunk * CH, CH)]],
                                  rows[b], sem_g[b]).wait()

        def fire_s(b):
            pltpu.async_copy(rows[b], shared.at[didx[b]], sem_s[b], add=True)

        def wait_s(b):
            pltpu.make_async_copy(rows[b], shared.at[didx[b]],
                                  sem_s[b]).wait()

        def wait_d(b):
            pltpu.make_async_copy(dst_hbm.at[wid, 0], didx[b],
                                  sem_d[b]).wait()

        for b in range(2):
            fire_d(b, b)
            fire_g(b, b)
        plsc.subcore_barrier()

        def step(ch, b, drain):
            wait_g(ch, b)
            wait_d(b)
            fire_s(b)
            b2 = (b + 2) % NBUF
            if drain:
                wait_s(b2)
            fire_d(ch + 2, b2)
            fire_g(ch + 2, b2)

        step(0, 0, False)
        step(1, 1, False)
        step(2, 2, True)
        step(3, 3, True)

        def body(io, carry):
            for b in range(NBUF):
                ch = NBUF * io + b
                wait_g(ch, b)
                wait_d(b)
                fire_s(b)
                b2 = (b + 2) % NBUF
                wait_s(b2)

                @pl.when(ch + 2 < NCH)
                def _():
                    fire_d(ch + 2, b2)
                    fire_g(ch + 2, b2)

            return carry

        lax.fori_loop(1, NCH // NBUF, body, 0)

        wait_s(2)
        wait_s(3)
        plsc.subcore_barrier()

        pltpu.sync_copy(shared.at[pl.ds(rstart, RPT)],
                        out_hbm.at[c].at[pl.ds(rstart, RPT)])

    return agg


_agg = _make_agg()


def _bn_stats(t):
    rid = lax.broadcasted_iota(jnp.int32, (NP, 1), 0)
    m = (rid < N).astype(jnp.float32)
    tm = t * m
    s1 = jnp.sum(tm, axis=0, keepdims=True)
    s2 = jnp.sum(tm * tm, axis=0, keepdims=True)
    mu = s1 / N
    var = s2 / N - mu * mu
    return m, mu, lax.rsqrt(var + BN_EPS)


def _dense_body(p_ref, w_ref, b_ref, g_ref, be_ref, o_ref):
    u = p_ref[0] + p_ref[1]
    t = jnp.dot(u, w_ref[...], preferred_element_type=jnp.float32) + b_ref[...]
    m, mu, rstd = _bn_stats(t)
    o_ref[...] = jnp.maximum(g_ref[...] * (t - mu) * rstd + be_ref[...],
                             0.0) * m


def _dense3_body(p_ref, w_ref, b_ref, g_ref, be_ref, w4_ref, b4_ref, o_ref):
    u = p_ref[0] + p_ref[1]
    t = jnp.dot(u, w_ref[...], preferred_element_type=jnp.float32) + b_ref[...]
    m, mu, rstd = _bn_stats(t)
    y = jnp.maximum(g_ref[...] * (t - mu) * rstd + be_ref[...], 0.0)
    o_ref[...] = (jnp.dot(y, w4_ref[...], preferred_element_type=jnp.float32)
                  + b4_ref[...])


def _dense(p, W, b, g, be):
    return pl.pallas_call(
        _dense_body,
        out_shape=jax.ShapeDtypeStruct((NP, D), jnp.float32),
    )(p, W, b.reshape(1, D), g.reshape(1, D), be.reshape(1, D))


def _dense3(p, W, b, g, be, W4, b4):
    return pl.pallas_call(
        _dense3_body,
        out_shape=jax.ShapeDtypeStruct((NP, D), jnp.float32),
    )(p, W, b.reshape(1, D), g.reshape(1, D), be.reshape(1, D),
      W4, b4.reshape(1, D))


def kernel(x, adj_t, W1, b1, g1, be1, W2, b2, g2, be2, W3, b3, g3, be3, W4, b4):
    npad = EPWP - EPW
    dpad = N + (jnp.arange(NW * npad, dtype=jnp.int32) % (NP - N - 8))
    src = jnp.concatenate(
        [adj_t[0].reshape(NW, EPW), dpad.reshape(NW, npad)], axis=1)
    dst = jnp.concatenate(
        [adj_t[1].reshape(NW, EPW), dpad.reshape(NW, npad)],
        axis=1).reshape(NW, NCH, CH)
    xp = jnp.pad(x, ((0, NP - N), (0, 0)))
    z = jnp.zeros((NP, D), jnp.float32)
    p = _agg(xp, z, src, dst)
    h = _dense(p, W1, b1, g1, be1)
    p = _agg(h, z, src, dst)
    h = _dense(p, W2, b2, g2, be2)
    p = _agg(h, z, src, dst)
    out = _dense3(p, W3, b3, g3, be3, W4, b4)
    return out[:N]
